# Initial kernel scaffold; baseline (speedup 1.0000x reference)
import functools

import jax
import jax.numpy as jnp
from jax import lax
from jax.experimental import pallas as pl
from jax.experimental.pallas import tpu as pltpu
from jax.experimental.pallas import tpu_sc as plsc

_N_USERS = 25000
_N_ITEMS = 25000
_N = _N_USERS + _N_ITEMS
_E = 800000
_D = 64
_H = _D // 2
_TILES = 16
_CH = 80
_EPT = _E // _TILES
_NCHUNK = _EPT // _CH
_RPT = _N // _TILES
_ZR = 125


def _sc_body(ego2, cols, dsts, vals, out, idx_v, dst_v, val_v, rows_v, zbuf,
             acc, sem):
    c = lax.axis_index("c")
    t = lax.axis_index("s")

    def zfill(i, _):
        zbuf[i, pl.ds(0, 16)] = jnp.zeros((16,), jnp.float32)
        zbuf[i, pl.ds(16, 16)] = jnp.zeros((16,), jnp.float32)
        return 0

    lax.fori_loop(0, _ZR, zfill, 0)

    def zcopy(i, _):
        pltpu.sync_copy(zbuf, acc.at[pl.ds(t * _RPT + i * _ZR, _ZR)])
        return 0

    lax.fori_loop(0, _RPT // _ZR, zcopy, 0)
    plsc.subcore_barrier()

    cN = c * _N

    def chunk(i, _):
        base = t * _EPT + i * _CH
        pltpu.sync_copy(cols.at[pl.ds(base, _CH)], idx_v)
        for j in range(_CH // 16):
            idx_v[pl.ds(j * 16, 16)] = idx_v[pl.ds(j * 16, 16)] + cN
        pltpu.async_copy(ego2.at[idx_v], rows_v, sem).wait()
        pltpu.sync_copy(vals.at[pl.ds(base, _CH)], val_v)
        pltpu.sync_copy(dsts.at[pl.ds(base, _CH)], dst_v)

        def scale(e, _):
            v = val_v[e]
            rows_v[e, pl.ds(0, 16)] = rows_v[e, pl.ds(0, 16)] * v
            rows_v[e, pl.ds(16, 16)] = rows_v[e, pl.ds(16, 16)] * v
            return 0

        lax.fori_loop(0, _CH, scale, 0)
        pltpu.sync_copy(rows_v, acc.at[dst_v], add=True)
        return 0

    lax.fori_loop(0, _NCHUNK, chunk, 0)
    plsc.subcore_barrier()
    pltpu.sync_copy(acc.at[pl.ds(t * _RPT, _RPT)],
                    out.at[c, pl.ds(t * _RPT, _RPT)])


_sc_segment = functools.partial(
    pl.kernel,
    out_type=jax.ShapeDtypeStruct((2, _N, _H), jnp.float32),
    mesh=plsc.VectorSubcoreMesh(core_axis_name="c", subcore_axis_name="s"),
    scratch_types=[
        pltpu.VMEM((_CH,), jnp.int32),
        pltpu.VMEM((_CH,), jnp.int32),
        pltpu.VMEM((_CH,), jnp.float32),
        pltpu.VMEM((_CH, _H), jnp.float32),
        pltpu.VMEM((_ZR, _H), jnp.float32),
        pltpu.VMEM_SHARED((_N, _H), jnp.float32),
        pltpu.SemaphoreType.DMA,
    ],
)(_sc_body)


_BR = 1000


def _tc_body(s_ref, ego_ref, acc_ref, w1_ref, b1_ref, w2_ref, b2_ref,
             eo_ref, e2_ref, acco_ref):
    side = jnp.concatenate([s_ref[0], s_ref[1]], axis=-1)
    ego = ego_ref[...]
    sum_emb = jnp.dot(side, w1_ref[...].T,
                      preferred_element_type=jnp.float32) + b1_ref[...]
    bi_emb = jnp.dot(ego * side, w2_ref[...].T,
                     preferred_element_type=jnp.float32) + b2_ref[...]
    x = sum_emb + bi_emb
    egon = jnp.where(x > 0, x, 0.2 * x)
    ssq = jnp.sum(egon * egon, axis=1, keepdims=True)
    nrm = jnp.sqrt(ssq)
    norm = egon / jnp.maximum(nrm, 1e-12)
    eo_ref[...] = egon
    e2_ref[0] = egon[:, :_H]
    e2_ref[1] = egon[:, _H:]
    acco_ref[...] = acc_ref[...] + norm


_tc_dense = pl.pallas_call(
    _tc_body,
    grid=(_N // _BR,),
    in_specs=[
        pl.BlockSpec((2, _BR, _H), lambda i: (0, i, 0)),
        pl.BlockSpec((_BR, _D), lambda i: (i, 0)),
        pl.BlockSpec((_BR, _D), lambda i: (i, 0)),
        pl.BlockSpec((_D, _D), lambda i: (0, 0)),
        pl.BlockSpec((1, _D), lambda i: (0, 0)),
        pl.BlockSpec((_D, _D), lambda i: (0, 0)),
        pl.BlockSpec((1, _D), lambda i: (0, 0)),
    ],
    out_specs=[
        pl.BlockSpec((_BR, _D), lambda i: (i, 0)),
        pl.BlockSpec((2, _BR, _H), lambda i: (0, i, 0)),
        pl.BlockSpec((_BR, _D), lambda i: (i, 0)),
    ],
    out_shape=[
        jax.ShapeDtypeStruct((_N, _D), jnp.float32),
        jax.ShapeDtypeStruct((2, _N, _H), jnp.float32),
        jax.ShapeDtypeStruct((_N, _D), jnp.float32),
    ],
)


def kernel(adj_indices, adj_values, user_emb, item_emb,
           W1_0, b1_0, W2_0, b2_0,
           W1_1, b1_1, W2_1, b2_1,
           W1_2, b1_2, W2_2, b2_2):
    weights = [
        (W1_0, b1_0, W2_0, b2_0),
        (W1_1, b1_1, W2_1, b2_1),
        (W1_2, b1_2, W2_2, b2_2),
    ]
    ego0 = jnp.concatenate([user_emb, item_emb], axis=0)
    dsts = adj_indices[0].astype(jnp.int32)
    cols = adj_indices[1].astype(jnp.int32)
    vals = adj_values.astype(jnp.float32)

    ego = ego0
    ego2 = jnp.concatenate([ego0[:, :_H], ego0[:, _H:]], axis=0)
    acc = ego0
    for (W1, b1, W2, b2) in weights:
        side2 = _sc_segment(ego2, cols, dsts, vals)
        ego, ego2s, acc = _tc_dense(side2, ego, acc,
                                    W1, b1.reshape(1, _D),
                                    W2, b2.reshape(1, _D))
        ego2 = ego2s.reshape(2 * _N, _H)
    all_emb = acc * 0.25
    return all_emb[:_N_USERS], all_emb[_N_USERS:]


# speedup vs baseline: 2.3787x; 2.3787x over previous
import functools

import jax
import jax.numpy as jnp
from jax import lax
from jax.experimental import pallas as pl
from jax.experimental.pallas import tpu as pltpu
from jax.experimental.pallas import tpu_sc as plsc

_N_USERS = 25000
_N_ITEMS = 25000
_N = _N_USERS + _N_ITEMS
_E = 800000
_D = 64
_H = _D // 2
_TILES = 16
_CH = 80
_EPT = _E // _TILES
_NCHUNK = _EPT // _CH
_NP = 50048
_RPT = _NP // _TILES
_ZR = 136


def _sc_body(ego2, cols, dsts, vals, out, idx_v, dst_v, val_v, rows_v, zbuf,
             acc, sem):
    c = lax.axis_index("c")
    t = lax.axis_index("s")

    def zfill(i, _):
        zbuf[i, pl.ds(0, 16)] = jnp.zeros((16,), jnp.float32)
        zbuf[i, pl.ds(16, 16)] = jnp.zeros((16,), jnp.float32)
        return 0

    lax.fori_loop(0, _ZR, zfill, 0)

    def zcopy(i, _):
        pltpu.sync_copy(zbuf, acc.at[pl.ds(t * _RPT + i * _ZR, _ZR)])
        return 0

    lax.fori_loop(0, _RPT // _ZR, zcopy, 0)
    plsc.subcore_barrier()

    cN = c * _N

    def chunk(i, _):
        base = t * _EPT + i * _CH
        pltpu.sync_copy(cols.at[pl.ds(base, _CH)], idx_v)
        for j in range(_CH // 16):
            idx_v[pl.ds(j * 16, 16)] = idx_v[pl.ds(j * 16, 16)] + cN
        pltpu.async_copy(ego2.at[idx_v], rows_v, sem).wait()
        pltpu.sync_copy(vals.at[pl.ds(base, _CH)], val_v)
        pltpu.sync_copy(dsts.at[pl.ds(base, _CH)], dst_v)

        def scale(g, _):
            vv = val_v[pl.ds(g * 16, 16)]
            for j in range(16):
                e = g * 16 + j
                v = vv[j]
                rows_v[e, pl.ds(0, 16)] = rows_v[e, pl.ds(0, 16)] * v
                rows_v[e, pl.ds(16, 16)] = rows_v[e, pl.ds(16, 16)] * v
            return 0

        lax.fori_loop(0, _CH // 16, scale, 0)
        pltpu.sync_copy(rows_v, acc.at[dst_v], add=True)
        return 0

    lax.fori_loop(0, _NCHUNK, chunk, 0)
    plsc.subcore_barrier()
    pltpu.sync_copy(acc.at[pl.ds(t * _RPT, _RPT)],
                    out.at[c, pl.ds(t * _RPT, _RPT)])


_sc_segment = functools.partial(
    pl.kernel,
    out_type=jax.ShapeDtypeStruct((2, _NP, _H), jnp.float32),
    mesh=plsc.VectorSubcoreMesh(core_axis_name="c", subcore_axis_name="s"),
    scratch_types=[
        pltpu.VMEM((_CH,), jnp.int32),
        pltpu.VMEM((_CH,), jnp.int32),
        pltpu.VMEM((_CH,), jnp.float32),
        pltpu.VMEM((_CH, _H), jnp.float32),
        pltpu.VMEM((_ZR, _H), jnp.float32),
        pltpu.VMEM_SHARED((_NP, _H), jnp.float32),
        pltpu.SemaphoreType.DMA,
    ],
    compiler_params=pltpu.CompilerParams(use_tc_tiling_on_sc=False),
)(_sc_body)


_BR = 1000


def _tc_body(s_ref, ego_ref, acc_ref, w1_ref, b1_ref, w2_ref, b2_ref,
             eo_ref, e2_ref, acco_ref):
    side = jnp.concatenate([s_ref[0], s_ref[1]], axis=-1)
    ego = ego_ref[...]
    sum_emb = jnp.dot(side, w1_ref[...].T,
                      preferred_element_type=jnp.float32) + b1_ref[...]
    bi_emb = jnp.dot(ego * side, w2_ref[...].T,
                     preferred_element_type=jnp.float32) + b2_ref[...]
    x = sum_emb + bi_emb
    egon = jnp.where(x > 0, x, 0.2 * x)
    ssq = jnp.sum(egon * egon, axis=1, keepdims=True)
    nrm = jnp.sqrt(ssq)
    norm = egon / jnp.maximum(nrm, 1e-12)
    eo_ref[...] = egon
    e2_ref[0] = egon[:, :_H]
    e2_ref[1] = egon[:, _H:]
    acco_ref[...] = acc_ref[...] + norm


_tc_dense = pl.pallas_call(
    _tc_body,
    grid=(_N // _BR,),
    in_specs=[
        pl.BlockSpec((2, _BR, _H), lambda i: (0, i, 0)),
        pl.BlockSpec((_BR, _D), lambda i: (i, 0)),
        pl.BlockSpec((_BR, _D), lambda i: (i, 0)),
        pl.BlockSpec((_D, _D), lambda i: (0, 0)),
        pl.BlockSpec((1, _D), lambda i: (0, 0)),
        pl.BlockSpec((_D, _D), lambda i: (0, 0)),
        pl.BlockSpec((1, _D), lambda i: (0, 0)),
    ],
    out_specs=[
        pl.BlockSpec((_BR, _D), lambda i: (i, 0)),
        pl.BlockSpec((2, _BR, _H), lambda i: (0, i, 0)),
        pl.BlockSpec((_BR, _D), lambda i: (i, 0)),
    ],
    out_shape=[
        jax.ShapeDtypeStruct((_N, _D), jnp.float32),
        jax.ShapeDtypeStruct((2, _N, _H), jnp.float32),
        jax.ShapeDtypeStruct((_N, _D), jnp.float32),
    ],
)


def kernel(adj_indices, adj_values, user_emb, item_emb,
           W1_0, b1_0, W2_0, b2_0,
           W1_1, b1_1, W2_1, b2_1,
           W1_2, b1_2, W2_2, b2_2):
    weights = [
        (W1_0, b1_0, W2_0, b2_0),
        (W1_1, b1_1, W2_1, b2_1),
        (W1_2, b1_2, W2_2, b2_2),
    ]
    ego0 = jnp.concatenate([user_emb, item_emb], axis=0)
    dsts = adj_indices[0].astype(jnp.int32)
    cols = adj_indices[1].astype(jnp.int32)
    vals = adj_values.astype(jnp.float32)

    ego = ego0
    ego2 = jnp.concatenate([ego0[:, :_H], ego0[:, _H:]], axis=0)
    acc = ego0
    for (W1, b1, W2, b2) in weights:
        side2 = _sc_segment(ego2, cols, dsts, vals)
        ego, ego2s, acc = _tc_dense(side2, ego, acc,
                                    W1, b1.reshape(1, _D),
                                    W2, b2.reshape(1, _D))
        ego2 = ego2s.reshape(2 * _N, _H)
    all_emb = acc * 0.25
    return all_emb[:_N_USERS], all_emb[_N_USERS:]
